# Initial kernel scaffold; baseline (speedup 1.0000x reference)
import jax
import jax.numpy as jnp
from jax.experimental import pallas as pl


def kernel(pred, target_coeffs):
    raise NotImplementedError("write your pallas kernel here")


# speedup vs baseline: 44.7121x; 44.7121x over previous
import math

import jax
import jax.numpy as jnp
import numpy as np
from jax.experimental import pallas as pl
from jax.experimental.pallas import tpu as pltpu

NLAT = 128
NLON = 256
LMAX = 50
MMAX = 50
N = 1024

_PI = math.pi


def _cc_quad_weights(n):
    tj = np.pi * np.arange(n) / (n - 1)
    x = np.cos(tj)
    Nn = n - 1
    w = np.zeros(n)
    for j in range(n):
        tmp = 0.0
        for k in range(1, Nn // 2 + 1):
            bk = 1.0 if 2 * k == Nn else 2.0
            tmp += bk / (4.0 * k * k - 1.0) * np.cos(2.0 * k * tj[j])
        wj = 1.0 - tmp
        wj = wj / Nn if (j == 0 or j == Nn) else 2.0 * wj / Nn
        w[j] = wj
    return x, w


def _legpoly(mmax, lmax, x):
    nlat = x.shape[0]
    pct = np.zeros((mmax, lmax, nlat))
    sint = np.sqrt(np.clip(1.0 - x * x, 0.0, None))
    pmm = np.full(nlat, math.sqrt(1.0 / (4.0 * math.pi)))
    for m in range(mmax):
        if m > 0:
            pmm = -math.sqrt((2.0 * m + 1.0) / (2.0 * m)) * sint * pmm
        if m < lmax:
            pct[m, m] = pmm
        if m + 1 < lmax:
            pct[m, m + 1] = math.sqrt(2.0 * m + 3.0) * x * pmm
        for l in range(m + 2, lmax):
            a = math.sqrt((4.0 * l * l - 1.0) / (l * l - m * m))
            b = math.sqrt((((l - 1.0) ** 2) - m * m) / (4.0 * (l - 1.0) ** 2 - 1.0))
            pct[m, l] = a * (x * pct[m, l - 1] - b * pct[m, l - 2])
    return pct


_COST, _WQ = _cc_quad_weights(NLAT)
_SHT_W = (_legpoly(MMAX, LMAX, _COST) * _WQ[None, None, :]).astype(np.float32)
_WT = np.ascontiguousarray(np.transpose(_SHT_W, (2, 1, 0)))
_j = np.arange(NLON)[:, None].astype(np.float64)
_m = np.arange(MMAX)[None, :].astype(np.float64)
_COS = np.cos(2.0 * np.pi * _j * _m / NLON).astype(np.float32)
_RW = np.exp(-((LMAX - np.arange(1, LMAX + 1)) ** 2) / (2.0 * LMAX ** 2)).astype(np.float32)[:, None]

QBLK = 1024
NBLK = (NLAT * NLON) // QBLK


def _knn_kernel(px_ref, py_ref, pz_ref, out_ref, sx_s, sy_s, ft_s):
    blk = pl.program_id(1)

    @pl.when(blk == 0)
    def _spherical():
        x = px_ref[0]
        y = py_ref[0]
        z = pz_ref[0]
        r = jnp.sqrt(x * x + y * y + z * z)
        rho = jnp.sqrt(y * y + z * z)

        def acos(v):
            return jnp.arctan2(jnp.sqrt((1.0 - v) * (1.0 + v)), v)

        theta = acos(jnp.clip(x / r, -1.0, 1.0))
        a = acos(jnp.clip(y / rho, -1.0, 1.0))
        phi = jnp.where(z < 0.0, 2.0 * _PI - a, a) - _PI
        sx_s[...] = theta
        sy_s[...] = phi
        ft_s[...] = r

    g = jax.lax.broadcasted_iota(jnp.int32, (QBLK, 1), 0) + blk * QBLK
    row = jax.lax.shift_right_logical(g, 8)
    col = jnp.bitwise_and(g, 255)
    tq = (row.astype(jnp.float32) / np.float32(NLAT)) * np.float32(_PI)
    pq = ((col.astype(jnp.float32) - np.float32(NLAT)) / np.float32(NLAT)) * np.float32(_PI)

    sx = sx_s[...]
    sy = sy_s[...]
    ft = ft_s[...]

    dx = tq - sx
    dy = pq - sy
    d2 = dx * dx + dy * dy

    inf = jnp.float32(jnp.inf)

    def take_min(d):
        m = jnp.min(d, axis=1, keepdims=True)
        eq = d == m
        f = jnp.min(jnp.where(eq, ft, inf), axis=1, keepdims=True)
        d_next = jnp.where(eq, inf, d)
        w = jnp.sqrt(jnp.maximum(m, 1e-12))
        return w, f, d_next

    w1, f1, d2 = take_min(d2)
    w2, f2, d2 = take_min(d2)
    w3, f3, _ = take_min(d2)

    interp = (w1 * f1 + w2 * f2 + w3 * f3) / (w1 + w2 + w3)
    out_ref[...] = interp.reshape(1, 1, QBLK)


def _sht_kernel(x_ref, t_ref, cos_ref, wt_ref, rw_ref, out_ref):
    x = x_ref[...]
    xr = jax.lax.dot(x, cos_ref[...], precision=jax.lax.Precision.HIGHEST,
                     preferred_element_type=jnp.float32)
    xr = xr * np.float32(2.0 * _PI / NLON)
    wt = wt_ref[...]
    rw = rw_ref[...]
    loss = jnp.float32(0.0)
    for b in range(2):
        xb = xr[b * NLAT:(b + 1) * NLAT]
        cb = jnp.sum(wt * xb[:, None, :], axis=0)
        resid = cb - t_ref[b]
        loss = loss + jnp.sum(resid * resid * rw)
    out_ref[...] = (loss * 0.5).reshape(1, 1)


def kernel(pred, target_coeffs):
    B = pred.shape[0]
    px = pred[:, :, 0].reshape(B, 1, N)
    py = pred[:, :, 1].reshape(B, 1, N)
    pz = pred[:, :, 2].reshape(B, 1, N)

    interp = pl.pallas_call(
        _knn_kernel,
        grid=(B, NBLK),
        in_specs=[
            pl.BlockSpec((1, 1, N), lambda b, i: (b, 0, 0)),
            pl.BlockSpec((1, 1, N), lambda b, i: (b, 0, 0)),
            pl.BlockSpec((1, 1, N), lambda b, i: (b, 0, 0)),
        ],
        out_specs=pl.BlockSpec((1, 1, QBLK), lambda b, i: (b * NBLK + i, 0, 0)),
        out_shape=jax.ShapeDtypeStruct((B * NBLK, 1, QBLK), jnp.float32),
        scratch_shapes=[
            pltpu.VMEM((1, N), jnp.float32),
            pltpu.VMEM((1, N), jnp.float32),
            pltpu.VMEM((1, N), jnp.float32),
        ],
    )(px, py, pz)

    x = interp.reshape(B * NLAT, NLON)
    loss = pl.pallas_call(
        _sht_kernel,
        out_shape=jax.ShapeDtypeStruct((1, 1), jnp.float32),
    )(x, target_coeffs, jnp.asarray(_COS), jnp.asarray(_WT), jnp.asarray(_RW))
    return loss[0, 0]
